# baseline (speedup 1.0000x reference)
import jax
import jax.numpy as jnp
from jax.experimental import pallas as pl
from jax.experimental.pallas import tpu as pltpu

_BM = 256
_NBUF = 4


def _gnn_ring_kernel(f_ref, w_ref, adj_hbm, o_ref, buf, sem):
    n = o_ref.shape[0]
    nblocks = n // _BM

    support = jnp.dot(f_ref[...], w_ref[...], preferred_element_type=jnp.float32)

    for t in range(_NBUF):
        pltpu.make_async_copy(
            adj_hbm.at[pl.ds(t * _BM, _BM), :], buf.at[t], sem.at[t]
        ).start()

    for t in range(nblocks):
        b = t % _NBUF
        pltpu.make_async_copy(
            adj_hbm.at[pl.ds(t * _BM, _BM), :], buf.at[b], sem.at[b]
        ).wait()
        o_ref[pl.ds(t * _BM, _BM), :] = jnp.maximum(
            jnp.dot(buf[b], support, preferred_element_type=jnp.float32), 0.0
        )
        nxt = t + _NBUF
        if nxt < nblocks:
            pltpu.make_async_copy(
                adj_hbm.at[pl.ds(nxt * _BM, _BM), :], buf.at[b], sem.at[b]
            ).start()


def kernel(features, adj, W):
    n, d_in = features.shape
    d_out = W.shape[1]
    return pl.pallas_call(
        _gnn_ring_kernel,
        in_specs=[
            pl.BlockSpec((n, d_in), lambda: (0, 0)),
            pl.BlockSpec((d_in, d_out), lambda: (0, 0)),
            pl.BlockSpec(memory_space=pltpu.MemorySpace.HBM),
        ],
        out_specs=pl.BlockSpec((n, d_out), lambda: (0, 0)),
        out_shape=jax.ShapeDtypeStruct((n, d_out), jnp.float32),
        scratch_shapes=[
            pltpu.VMEM((_NBUF, _BM, n), jnp.float32),
            pltpu.SemaphoreType.DMA((_NBUF,)),
        ],
    )(features, W, adj)


# speedup vs baseline: 1.3588x; 1.3588x over previous
import jax
import jax.numpy as jnp
from jax import lax
from jax.experimental import pallas as pl
from jax.experimental.pallas import tpu as pltpu


def _gnn_kernel(ft_ref, w_ref, adj_ref, ot_ref, st_ref):
    @pl.when(pl.program_id(0) == 0)
    def _():
        st_ref[...] = lax.dot_general(
            w_ref[...],
            ft_ref[...],
            (((0,), (0,)), ((), ())),
            preferred_element_type=jnp.float32,
        )

    ot_ref[...] = jnp.maximum(
        lax.dot_general(
            st_ref[...],
            adj_ref[...],
            (((1,), (1,)), ((), ())),
            preferred_element_type=jnp.float32,
        ),
        0.0,
    )


def kernel(features, adj, W):
    n, d_in = features.shape
    d_out = W.shape[1]
    bm = 512
    grid = (n // bm,)
    out_t = pl.pallas_call(
        _gnn_kernel,
        grid=grid,
        in_specs=[
            pl.BlockSpec((d_in, n), lambda i: (0, 0)),
            pl.BlockSpec((d_in, d_out), lambda i: (0, 0)),
            pl.BlockSpec((bm, n), lambda i: (i, 0)),
        ],
        out_specs=pl.BlockSpec((d_out, bm), lambda i: (0, i)),
        out_shape=jax.ShapeDtypeStruct((d_out, n), jnp.float32),
        scratch_shapes=[pltpu.VMEM((d_out, n), jnp.float32)],
    )(features.T, W, adj)
    return out_t.T
